# Initial kernel scaffold; baseline (speedup 1.0000x reference)
import jax
import jax.numpy as jnp
from jax.experimental import pallas as pl


def kernel(x, patch_W, patch_b, pos_embed, ln1_g, ln1_b, qkv_w, qkv_b, out_w, out_b, router_w, router_b, eW1, eb1, eW2, eb2, ln2_g, ln2_b, head_w, head_b):
    raise NotImplementedError("write your pallas kernel here")


# speedup vs baseline: 2.1227x; 2.1227x over previous
import functools

import jax
import jax.numpy as jnp
from jax.experimental import pallas as pl
from jax.experimental.pallas import tpu as pltpu

B = 32; C = 3; IMG = 224; P = 16; N = (IMG // P) ** 2; E = 384; H = 6; DH = E // H
HID = int(E * 4.0); NE = 8; K = 2; NC = 1000
T = B * N
TILE = 512
NT = 32
S = NT * TILE


def _ln(x, g, b):
    m = jnp.mean(x, axis=-1, keepdims=True)
    v = jnp.mean((x - m) ** 2, axis=-1, keepdims=True)
    return (x - m) / jnp.sqrt(v + 1e-5) * g + b


def _attn_body(p_ref, pos_ref, pw_ref, pb_ref, g1_ref, b1_ref, qkvw_ref,
               qkvb_ref, ow_ref, ob_ref, rw_ref, rb_ref, z_ref, lg_ref):
    p = p_ref[0]
    z = jnp.dot(p, pw_ref[...], preferred_element_type=jnp.float32)
    z = z + pb_ref[...] + pos_ref[0]
    y = _ln(z, g1_ref[...], b1_ref[...])
    qkv = jnp.dot(y, qkvw_ref[...], preferred_element_type=jnp.float32)
    qkv = qkv + qkvb_ref[...]
    aos = []
    for h in range(H):
        q = qkv[:, h * DH:(h + 1) * DH]
        k = qkv[:, E + h * DH:E + (h + 1) * DH]
        v = qkv[:, 2 * E + h * DH:2 * E + (h + 1) * DH]
        s = jax.lax.dot_general(q, k, (((1,), (1,)), ((), ())),
                                preferred_element_type=jnp.float32)
        s = s * (1.0 / (DH ** 0.5))
        s = s - jnp.max(s, axis=-1, keepdims=True)
        es = jnp.exp(s)
        att = es / jnp.sum(es, axis=-1, keepdims=True)
        aos.append(jnp.dot(att, v, preferred_element_type=jnp.float32))
    ao = jnp.concatenate(aos, axis=-1)
    z2 = z + (jnp.dot(ao, ow_ref[...], preferred_element_type=jnp.float32) + ob_ref[...])
    z_ref[0] = z2
    lg_ref[0] = jnp.dot(z2, rw_ref[...], preferred_element_type=jnp.float32) + rb_ref[...]


def _gelu(x):
    return 0.5 * x * (1.0 + jax.lax.erf(x * (2.0 ** -0.5)))


def _moe_body(ept_ref, xs_ref, w1_ref, b1_ref, w2_ref, b2_ref, o_ref):
    x = xs_ref[...]
    h = jax.lax.dot_general(x, w1_ref[0], (((1,), (1,)), ((), ())),
                            preferred_element_type=jnp.float32)
    h = _gelu(h + b1_ref[0])
    y = jax.lax.dot_general(h, w2_ref[0], (((1,), (1,)), ((), ())),
                            preferred_element_type=jnp.float32)
    o_ref[...] = y + b2_ref[0]


def _head_body(ye_ref, yo_ref, g_ref, b_ref, hw_ref, hb_ref, o_ref):
    m = (ye_ref[0] + yo_ref[0]) * (1.0 / K)
    z = _ln(m, g_ref[...], b_ref[...])
    pooled = jnp.mean(z, axis=0, keepdims=True)
    o_ref[0] = jnp.dot(pooled, hw_ref[...], preferred_element_type=jnp.float32) + hb_ref[...]


def kernel(x, patch_W, patch_b, pos_embed, ln1_g, ln1_b, qkv_w, qkv_b, out_w,
           out_b, router_w, router_b, eW1, eb1, eW2, eb2, ln2_g, ln2_b,
           head_w, head_b):
    f32 = jnp.float32
    patches = x.reshape(B, C, N // (IMG // P), P, IMG // P, P) \
               .transpose(0, 2, 4, 1, 3, 5).reshape(B, N, C * P * P)
    pwT = patch_W.reshape(E, C * P * P).T
    qkvwT = qkv_w.T
    owT = out_w.T
    rwT = router_w.T
    hwT = head_w.T
    row2 = lambda a: a.reshape(1, -1)

    z, logits = pl.pallas_call(
        _attn_body,
        grid=(B,),
        in_specs=[
            pl.BlockSpec((1, N, C * P * P), lambda i: (i, 0, 0)),
            pl.BlockSpec((1, N, E), lambda i: (0, 0, 0)),
            pl.BlockSpec((C * P * P, E), lambda i: (0, 0)),
            pl.BlockSpec((1, E), lambda i: (0, 0)),
            pl.BlockSpec((1, E), lambda i: (0, 0)),
            pl.BlockSpec((1, E), lambda i: (0, 0)),
            pl.BlockSpec((E, 3 * E), lambda i: (0, 0)),
            pl.BlockSpec((1, 3 * E), lambda i: (0, 0)),
            pl.BlockSpec((E, E), lambda i: (0, 0)),
            pl.BlockSpec((1, E), lambda i: (0, 0)),
            pl.BlockSpec((E, NE), lambda i: (0, 0)),
            pl.BlockSpec((1, NE), lambda i: (0, 0)),
        ],
        out_specs=[
            pl.BlockSpec((1, N, E), lambda i: (i, 0, 0)),
            pl.BlockSpec((1, N, NE), lambda i: (i, 0, 0)),
        ],
        out_shape=[
            jax.ShapeDtypeStruct((B, N, E), f32),
            jax.ShapeDtypeStruct((B, N, NE), f32),
        ],
    )(patches, pos_embed, pwT, row2(patch_b), row2(ln1_g), row2(ln1_b),
      qkvwT, row2(qkv_b), owT, row2(out_b), rwT, row2(router_b))

    _, topk_idx = jax.lax.top_k(logits.reshape(T, NE), K)
    ef = topk_idx.reshape(-1)
    oh = (ef[:, None] == jnp.arange(NE, dtype=ef.dtype)).astype(jnp.int32)
    rank = jnp.take_along_axis(jnp.cumsum(oh, axis=0) - oh, ef[:, None], 1)[:, 0]
    counts = jnp.sum(oh, axis=0)
    padded = ((counts + TILE - 1) // TILE) * TILE
    cum = jnp.cumsum(padded)
    offs = cum - padded
    dest = offs[ef] + rank
    ept = jnp.searchsorted(cum, jnp.arange(NT, dtype=jnp.int32) * TILE,
                           side="right").astype(jnp.int32)
    ept = jnp.minimum(ept, NE - 1)

    z_flat = z.reshape(T, E)
    tok_per_slot = jnp.zeros((S,), jnp.int32).at[dest].set(
        jnp.arange(T * K, dtype=jnp.int32) // K)
    zs = z_flat[tok_per_slot]

    w1b = eW1
    w2b = eW2
    ys = pl.pallas_call(
        _moe_body,
        grid_spec=pltpu.PrefetchScalarGridSpec(
            num_scalar_prefetch=1,
            grid=(NT,),
            in_specs=[
                pl.BlockSpec((TILE, E), lambda t, ept: (t, 0)),
                pl.BlockSpec((1, HID, E), lambda t, ept: (ept[t], 0, 0)),
                pl.BlockSpec((1, 1, HID), lambda t, ept: (ept[t], 0, 0)),
                pl.BlockSpec((1, E, HID), lambda t, ept: (ept[t], 0, 0)),
                pl.BlockSpec((1, 1, E), lambda t, ept: (ept[t], 0, 0)),
            ],
            out_specs=pl.BlockSpec((TILE, E), lambda t, ept: (t, 0)),
        ),
        out_shape=jax.ShapeDtypeStruct((S, E), f32),
    )(ept, zs, w1b, eb1.reshape(NE, 1, HID), w2b, eb2.reshape(NE, 1, E))

    destE = dest[0::2]
    destO = dest[1::2]
    ysE = ys[destE].reshape(B, N, E)
    ysO = ys[destO].reshape(B, N, E)
    out = pl.pallas_call(
        _head_body,
        grid=(B,),
        in_specs=[
            pl.BlockSpec((1, N, E), lambda i: (i, 0, 0)),
            pl.BlockSpec((1, N, E), lambda i: (i, 0, 0)),
            pl.BlockSpec((1, E), lambda i: (0, 0)),
            pl.BlockSpec((1, E), lambda i: (0, 0)),
            pl.BlockSpec((E, NC), lambda i: (0, 0)),
            pl.BlockSpec((1, NC), lambda i: (0, 0)),
        ],
        out_specs=pl.BlockSpec((1, 1, NC), lambda i: (i, 0, 0)),
        out_shape=jax.ShapeDtypeStruct((B, 1, NC), f32),
    )(ysE, ysO, row2(ln2_g), row2(ln2_b), hwT, row2(head_b))
    return out.reshape(B, NC)


# speedup vs baseline: 2.6163x; 1.2325x over previous
import functools

import jax
import jax.numpy as jnp
from jax.experimental import pallas as pl
from jax.experimental.pallas import tpu as pltpu

B = 32; C = 3; IMG = 224; P = 16; N = (IMG // P) ** 2; E = 384; H = 6; DH = E // H
HID = int(E * 4.0); NE = 8; K = 2; NC = 1000
T = B * N
TILE = 512
NT = 32
S = NT * TILE


def _ln(x, g, b):
    m = jnp.mean(x, axis=-1, keepdims=True)
    v = jnp.mean((x - m) ** 2, axis=-1, keepdims=True)
    return (x - m) / jnp.sqrt(v + 1e-5) * g + b


def _attn_body(p_ref, pos_ref, pw_ref, pb_ref, g1_ref, b1_ref, qkvw_ref,
               qkvb_ref, ow_ref, ob_ref, rw_ref, rb_ref, z_ref, lg_ref):
    p = p_ref[0]
    z = jnp.dot(p, pw_ref[...], preferred_element_type=jnp.float32)
    z = z + pb_ref[...] + pos_ref[0]
    y = _ln(z, g1_ref[...], b1_ref[...])
    qkv = jnp.dot(y, qkvw_ref[...], preferred_element_type=jnp.float32)
    qkv = qkv + qkvb_ref[...]
    aos = []
    for h in range(H):
        q = qkv[:, h * DH:(h + 1) * DH]
        k = qkv[:, E + h * DH:E + (h + 1) * DH]
        v = qkv[:, 2 * E + h * DH:2 * E + (h + 1) * DH]
        s = jax.lax.dot_general(q, k, (((1,), (1,)), ((), ())),
                                preferred_element_type=jnp.float32)
        s = s * (1.0 / (DH ** 0.5))
        s = s - jnp.max(s, axis=-1, keepdims=True)
        es = jnp.exp(s)
        att = es / jnp.sum(es, axis=-1, keepdims=True)
        aos.append(jnp.dot(att, v, preferred_element_type=jnp.float32))
    ao = jnp.concatenate(aos, axis=-1)
    z2 = z + (jnp.dot(ao, ow_ref[...], preferred_element_type=jnp.float32) + ob_ref[...])
    z_ref[0] = z2
    lg_ref[0] = jnp.dot(z2, rw_ref[...], preferred_element_type=jnp.float32) + rb_ref[...]


def _gelu(x):
    return 0.5 * x * (1.0 + jax.lax.erf(x * (2.0 ** -0.5)))


def _moe_body(ept_ref, xs_ref, w1_ref, b1_ref, w2_ref, b2_ref, o_ref):
    x = xs_ref[...]
    h = jax.lax.dot_general(x, w1_ref[0], (((1,), (1,)), ((), ())),
                            preferred_element_type=jnp.float32)
    h = _gelu(h + b1_ref[0])
    y = jax.lax.dot_general(h, w2_ref[0], (((1,), (1,)), ((), ())),
                            preferred_element_type=jnp.float32)
    o_ref[...] = y + b2_ref[0]


def _head_body(ye_ref, yo_ref, g_ref, b_ref, hw_ref, hb_ref, o_ref):
    m = (ye_ref[0] + yo_ref[0]) * (1.0 / K)
    z = _ln(m, g_ref[...], b_ref[...])
    pooled = jnp.mean(z, axis=0, keepdims=True)
    o_ref[0] = jnp.dot(pooled, hw_ref[...], preferred_element_type=jnp.float32) + hb_ref[...]


CH = 112
NCH = T // CH
NW = 32


def _sc_mesh():
    from jax.experimental.pallas import tpu_sc as plsc
    return plsc.VectorSubcoreMesh(core_axis_name="c", subcore_axis_name="s")


def _sc_disperse(z_flat, destE2, destO2):
    @functools.partial(
        pl.kernel,
        mesh=_sc_mesh(),
        out_type=jax.ShapeDtypeStruct((S, E), jnp.float32),
        scratch_types=[
            pltpu.VMEM((CH,), jnp.int32),
            pltpu.VMEM((CH, E), jnp.float32),
            pltpu.SemaphoreType.DMA,
        ],
    )
    def disperse(z_hbm, de_hbm, do_hbm, zs_hbm, idx_v, buf_v, sem):
        w = jax.lax.axis_index("s") * 2 + jax.lax.axis_index("c")
        for j in range(2):
            ci = w + NW * j

            @pl.when(ci < NCH)
            def _():
                tok0 = ci * CH
                pltpu.sync_copy(z_hbm.at[pl.ds(tok0, CH)], buf_v)
                pltpu.sync_copy(de_hbm.at[pl.ds(tok0, CH)], idx_v)
                pltpu.async_copy(buf_v, zs_hbm.at[idx_v], sem).wait()
                pltpu.sync_copy(do_hbm.at[pl.ds(tok0, CH)], idx_v)
                pltpu.async_copy(buf_v, zs_hbm.at[idx_v], sem).wait()

    return disperse(z_flat, destE2, destO2)


def _sc_combine(ys, destE2, destO2):
    @functools.partial(
        pl.kernel,
        mesh=_sc_mesh(),
        out_type=[
            jax.ShapeDtypeStruct((T, E), jnp.float32),
            jax.ShapeDtypeStruct((T, E), jnp.float32),
        ],
        scratch_types=[
            pltpu.VMEM((CH,), jnp.int32),
            pltpu.VMEM((CH, E), jnp.float32),
            pltpu.VMEM((CH, E), jnp.float32),
            pltpu.SemaphoreType.DMA,
            pltpu.SemaphoreType.DMA,
        ],
    )
    def combine(ys_hbm, de_hbm, do_hbm, ye_hbm, yo_hbm, idx_v, bufe_v,
                bufo_v, seme, semo):
        w = jax.lax.axis_index("s") * 2 + jax.lax.axis_index("c")
        for j in range(2):
            ci = w + NW * j

            @pl.when(ci < NCH)
            def _():
                tok0 = ci * CH
                pltpu.sync_copy(de_hbm.at[pl.ds(tok0, CH)], idx_v)
                pltpu.async_copy(ys_hbm.at[idx_v], bufe_v, seme).wait()
                pltpu.sync_copy(do_hbm.at[pl.ds(tok0, CH)], idx_v)
                pltpu.async_copy(ys_hbm.at[idx_v], bufo_v, semo).wait()
                pltpu.sync_copy(bufe_v, ye_hbm.at[pl.ds(tok0, CH)])
                pltpu.sync_copy(bufo_v, yo_hbm.at[pl.ds(tok0, CH)])

    return combine(ys, destE2, destO2)


def kernel(x, patch_W, patch_b, pos_embed, ln1_g, ln1_b, qkv_w, qkv_b, out_w,
           out_b, router_w, router_b, eW1, eb1, eW2, eb2, ln2_g, ln2_b,
           head_w, head_b):
    f32 = jnp.float32
    patches = x.reshape(B, C, N // (IMG // P), P, IMG // P, P) \
               .transpose(0, 2, 4, 1, 3, 5).reshape(B, N, C * P * P)
    pwT = patch_W.reshape(E, C * P * P).T
    qkvwT = qkv_w.T
    owT = out_w.T
    rwT = router_w.T
    hwT = head_w.T
    row2 = lambda a: a.reshape(1, -1)

    z, logits = pl.pallas_call(
        _attn_body,
        grid=(B,),
        in_specs=[
            pl.BlockSpec((1, N, C * P * P), lambda i: (i, 0, 0)),
            pl.BlockSpec((1, N, E), lambda i: (0, 0, 0)),
            pl.BlockSpec((C * P * P, E), lambda i: (0, 0)),
            pl.BlockSpec((1, E), lambda i: (0, 0)),
            pl.BlockSpec((1, E), lambda i: (0, 0)),
            pl.BlockSpec((1, E), lambda i: (0, 0)),
            pl.BlockSpec((E, 3 * E), lambda i: (0, 0)),
            pl.BlockSpec((1, 3 * E), lambda i: (0, 0)),
            pl.BlockSpec((E, E), lambda i: (0, 0)),
            pl.BlockSpec((1, E), lambda i: (0, 0)),
            pl.BlockSpec((E, NE), lambda i: (0, 0)),
            pl.BlockSpec((1, NE), lambda i: (0, 0)),
        ],
        out_specs=[
            pl.BlockSpec((1, N, E), lambda i: (i, 0, 0)),
            pl.BlockSpec((1, N, NE), lambda i: (i, 0, 0)),
        ],
        out_shape=[
            jax.ShapeDtypeStruct((B, N, E), f32),
            jax.ShapeDtypeStruct((B, N, NE), f32),
        ],
    )(patches, pos_embed, pwT, row2(patch_b), row2(ln1_g), row2(ln1_b),
      qkvwT, row2(qkv_b), owT, row2(out_b), rwT, row2(router_b))

    _, topk_idx = jax.lax.top_k(logits.reshape(T, NE), K)
    ef = topk_idx.reshape(-1)
    oh = (ef[:, None] == jnp.arange(NE, dtype=ef.dtype)).astype(jnp.int32)
    rank = jnp.take_along_axis(jnp.cumsum(oh, axis=0) - oh, ef[:, None], 1)[:, 0]
    counts = jnp.sum(oh, axis=0)
    padded = ((counts + TILE - 1) // TILE) * TILE
    cum = jnp.cumsum(padded)
    offs = cum - padded
    dest = offs[ef] + rank
    ept = jnp.searchsorted(cum, jnp.arange(NT, dtype=jnp.int32) * TILE,
                           side="right").astype(jnp.int32)
    ept = jnp.minimum(ept, NE - 1)

    z_flat = z.reshape(T, E)
    destE2 = dest[0::2]
    destO2 = dest[1::2]
    zs = _sc_disperse(z_flat, destE2, destO2)

    w1b = eW1
    w2b = eW2
    ys = pl.pallas_call(
        _moe_body,
        grid_spec=pltpu.PrefetchScalarGridSpec(
            num_scalar_prefetch=1,
            grid=(NT,),
            in_specs=[
                pl.BlockSpec((TILE, E), lambda t, ept: (t, 0)),
                pl.BlockSpec((1, HID, E), lambda t, ept: (ept[t], 0, 0)),
                pl.BlockSpec((1, 1, HID), lambda t, ept: (ept[t], 0, 0)),
                pl.BlockSpec((1, E, HID), lambda t, ept: (ept[t], 0, 0)),
                pl.BlockSpec((1, 1, E), lambda t, ept: (ept[t], 0, 0)),
            ],
            out_specs=pl.BlockSpec((TILE, E), lambda t, ept: (t, 0)),
        ),
        out_shape=jax.ShapeDtypeStruct((S, E), f32),
    )(ept, zs, w1b, eb1.reshape(NE, 1, HID), w2b, eb2.reshape(NE, 1, E))

    ye, yo = _sc_combine(ys, destE2, destO2)
    ysE = ye.reshape(B, N, E)
    ysO = yo.reshape(B, N, E)
    out = pl.pallas_call(
        _head_body,
        grid=(B,),
        in_specs=[
            pl.BlockSpec((1, N, E), lambda i: (i, 0, 0)),
            pl.BlockSpec((1, N, E), lambda i: (i, 0, 0)),
            pl.BlockSpec((1, E), lambda i: (0, 0)),
            pl.BlockSpec((1, E), lambda i: (0, 0)),
            pl.BlockSpec((E, NC), lambda i: (0, 0)),
            pl.BlockSpec((1, NC), lambda i: (0, 0)),
        ],
        out_specs=pl.BlockSpec((1, 1, NC), lambda i: (i, 0, 0)),
        out_shape=jax.ShapeDtypeStruct((B, 1, NC), f32),
    )(ysE, ysO, row2(ln2_g), row2(ln2_b), hwT, row2(head_b))
    return out.reshape(B, NC)


# speedup vs baseline: 2.7431x; 1.0485x over previous
import functools

import jax
import jax.numpy as jnp
from jax.experimental import pallas as pl
from jax.experimental.pallas import tpu as pltpu

B = 32; C = 3; IMG = 224; P = 16; N = (IMG // P) ** 2; E = 384; H = 6; DH = E // H
HID = int(E * 4.0); NE = 8; K = 2; NC = 1000
T = B * N
TILE = 512
NT = 32
S = NT * TILE


def _ln(x, g, b):
    m = jnp.mean(x, axis=-1, keepdims=True)
    v = jnp.mean((x - m) ** 2, axis=-1, keepdims=True)
    return (x - m) / jnp.sqrt(v + 1e-5) * g + b


def _attn_body(p_ref, pos_ref, pw_ref, pb_ref, g1_ref, b1_ref, qkvw_ref,
               qkvb_ref, ow_ref, ob_ref, rw_ref, rb_ref, z_ref, lg_ref):
    p = p_ref[0]
    z = jnp.dot(p, pw_ref[...], preferred_element_type=jnp.float32)
    z = z + pb_ref[...] + pos_ref[0]
    y = _ln(z, g1_ref[...], b1_ref[...])
    qkv = jnp.dot(y, qkvw_ref[...], preferred_element_type=jnp.float32)
    qkv = qkv + qkvb_ref[...]
    aos = []
    for h in range(H):
        q = qkv[:, h * DH:(h + 1) * DH]
        k = qkv[:, E + h * DH:E + (h + 1) * DH]
        v = qkv[:, 2 * E + h * DH:2 * E + (h + 1) * DH]
        s = jax.lax.dot_general(q, k, (((1,), (1,)), ((), ())),
                                preferred_element_type=jnp.float32)
        s = s * (1.0 / (DH ** 0.5))
        s = s - jnp.max(s, axis=-1, keepdims=True)
        es = jnp.exp(s)
        att = es / jnp.sum(es, axis=-1, keepdims=True)
        aos.append(jnp.dot(att, v, preferred_element_type=jnp.float32))
    ao = jnp.concatenate(aos, axis=-1)
    z2 = z + (jnp.dot(ao, ow_ref[...], preferred_element_type=jnp.float32) + ob_ref[...])
    z_ref[0] = z2
    lg_ref[0] = jnp.dot(z2, rw_ref[...], preferred_element_type=jnp.float32) + rb_ref[...]


def _gelu(x):
    return 0.5 * x * (1.0 + jax.lax.erf(x * (2.0 ** -0.5)))


def _moe_body(ept_ref, xs_ref, w1_ref, b1_ref, w2_ref, b2_ref, o_ref):
    x = xs_ref[...].astype(jnp.bfloat16)
    h = jax.lax.dot_general(x, w1_ref[0], (((1,), (1,)), ((), ())),
                            preferred_element_type=jnp.float32)
    h = _gelu(h + b1_ref[0])
    y = jax.lax.dot_general(h.astype(jnp.bfloat16), w2_ref[0],
                            (((1,), (1,)), ((), ())),
                            preferred_element_type=jnp.float32)
    o_ref[...] = y + b2_ref[0]


def _route_body(lg_ref, d0_ref, d1_ref, ept_ref):
    lg = lg_ref[...]
    i8 = jax.lax.broadcasted_iota(jnp.int32, (T, NE), 1)
    v1 = jnp.max(lg, axis=1, keepdims=True)
    idx1 = jnp.min(jnp.where(lg == v1, i8, NE), axis=1, keepdims=True)
    lm = jnp.where(i8 == idx1, -3.4e38, lg)
    v2 = jnp.max(lm, axis=1, keepdims=True)
    idx2 = jnp.min(jnp.where(lm == v2, i8, NE), axis=1, keepdims=True)
    oh1 = (i8 == idx1).astype(jnp.float32)
    oh2 = (i8 == idx2).astype(jnp.float32)
    oh = oh1 + oh2
    r = jax.lax.broadcasted_iota(jnp.int32, (128, 128), 0)
    c = jax.lax.broadcasted_iota(jnp.int32, (128, 128), 1)
    Lx = (c < r).astype(jnp.float32)
    carry = jnp.zeros((1, NE), jnp.float32)
    cums = []
    for i in range(T // 128):
        blk = oh[i * 128:(i + 1) * 128, :]
        cums.append(jnp.dot(Lx, blk, preferred_element_type=jnp.float32) + carry)
        carry = carry + jnp.sum(blk, axis=0, keepdims=True)
    cum = jnp.concatenate(cums, axis=0)
    ci = carry.astype(jnp.int32)
    padded = jnp.bitwise_and(ci + (TILE - 1), ~(TILE - 1))
    pf = padded.astype(jnp.float32)
    er = jax.lax.broadcasted_iota(jnp.int32, (NE, NE), 0)
    ec = jax.lax.broadcasted_iota(jnp.int32, (NE, NE), 1)
    U8 = (er < ec).astype(jnp.float32)
    offs = jnp.dot(pf, U8, preferred_element_type=jnp.float32)
    pos = offs + cum
    d0_ref[...] = jnp.sum(oh1 * pos, axis=1, keepdims=True).astype(jnp.int32)
    d1_ref[...] = jnp.sum(oh2 * pos, axis=1, keepdims=True).astype(jnp.int32)
    cum8 = offs + pf
    t0 = jax.lax.broadcasted_iota(jnp.int32, (NT, NE), 0).astype(jnp.float32) * TILE
    ept = jnp.sum((cum8 <= t0).astype(jnp.int32), axis=1, keepdims=True)
    ept_ref[...] = jnp.minimum(ept, NE - 1)


IPB = 8


def _head_body(ye_ref, yo_ref, g_ref, b_ref, hw_ref, hb_ref, o_ref):
    m = (ye_ref[...] + yo_ref[...]) * (1.0 / K)
    z = _ln(m, g_ref[...].reshape(1, 1, E), b_ref[...].reshape(1, 1, E))
    pooled = jnp.mean(z, axis=1)
    o_ref[...] = jnp.dot(pooled, hw_ref[...], preferred_element_type=jnp.float32) + hb_ref[...]


CH = 112
NCH = T // CH
NW = 32


def _sc_mesh():
    from jax.experimental.pallas import tpu_sc as plsc
    return plsc.VectorSubcoreMesh(core_axis_name="c", subcore_axis_name="s")


def _sc_disperse(z_flat, destE2, destO2):
    @functools.partial(
        pl.kernel,
        mesh=_sc_mesh(),
        out_type=jax.ShapeDtypeStruct((S, E), jnp.float32),
        scratch_types=[
            pltpu.VMEM((CH,), jnp.int32),
            pltpu.VMEM((CH,), jnp.int32),
            pltpu.VMEM((CH, E), jnp.float32),
            pltpu.SemaphoreType.DMA,
            pltpu.SemaphoreType.DMA,
        ],
    )
    def disperse(z_hbm, de_hbm, do_hbm, zs_hbm, idxe_v, idxo_v, buf_v,
                 seme, semo):
        w = jax.lax.axis_index("s") * 2 + jax.lax.axis_index("c")
        for j in range(2):
            ci = w + NW * j

            @pl.when(ci < NCH)
            def _():
                tok0 = ci * CH
                pltpu.sync_copy(z_hbm.at[pl.ds(tok0, CH)], buf_v)
                pltpu.sync_copy(de_hbm.at[pl.ds(tok0, CH)], idxe_v)
                pltpu.sync_copy(do_hbm.at[pl.ds(tok0, CH)], idxo_v)
                ce = pltpu.async_copy(buf_v, zs_hbm.at[idxe_v], seme)
                co = pltpu.async_copy(buf_v, zs_hbm.at[idxo_v], semo)
                ce.wait()
                co.wait()

    return disperse(z_flat, destE2, destO2)


def _sc_combine(ys, destE2, destO2):
    @functools.partial(
        pl.kernel,
        mesh=_sc_mesh(),
        out_type=[
            jax.ShapeDtypeStruct((T, E), jnp.float32),
            jax.ShapeDtypeStruct((T, E), jnp.float32),
        ],
        scratch_types=[
            pltpu.VMEM((CH,), jnp.int32),
            pltpu.VMEM((CH,), jnp.int32),
            pltpu.VMEM((CH, E), jnp.float32),
            pltpu.VMEM((CH, E), jnp.float32),
            pltpu.SemaphoreType.DMA,
            pltpu.SemaphoreType.DMA,
        ],
    )
    def combine(ys_hbm, de_hbm, do_hbm, ye_hbm, yo_hbm, idxe_v, idxo_v,
                bufe_v, bufo_v, seme, semo):
        w = jax.lax.axis_index("s") * 2 + jax.lax.axis_index("c")
        for j in range(2):
            ci = w + NW * j

            @pl.when(ci < NCH)
            def _():
                tok0 = ci * CH
                pltpu.sync_copy(de_hbm.at[pl.ds(tok0, CH)], idxe_v)
                pltpu.sync_copy(do_hbm.at[pl.ds(tok0, CH)], idxo_v)
                ce = pltpu.async_copy(ys_hbm.at[idxe_v], bufe_v, seme)
                co = pltpu.async_copy(ys_hbm.at[idxo_v], bufo_v, semo)
                ce.wait()
                co.wait()
                pltpu.sync_copy(bufe_v, ye_hbm.at[pl.ds(tok0, CH)])
                pltpu.sync_copy(bufo_v, yo_hbm.at[pl.ds(tok0, CH)])

    return combine(ys, destE2, destO2)


def kernel(x, patch_W, patch_b, pos_embed, ln1_g, ln1_b, qkv_w, qkv_b, out_w,
           out_b, router_w, router_b, eW1, eb1, eW2, eb2, ln2_g, ln2_b,
           head_w, head_b):
    f32 = jnp.float32
    patches = x.reshape(B, C, N // (IMG // P), P, IMG // P, P) \
               .transpose(0, 2, 4, 1, 3, 5).reshape(B, N, C * P * P)
    pwT = patch_W.reshape(E, C * P * P).T
    qkvwT = qkv_w.T
    owT = out_w.T
    rwT = router_w.T
    hwT = head_w.T
    row2 = lambda a: a.reshape(1, -1)

    z, logits = pl.pallas_call(
        _attn_body,
        grid=(B,),
        in_specs=[
            pl.BlockSpec((1, N, C * P * P), lambda i: (i, 0, 0)),
            pl.BlockSpec((1, N, E), lambda i: (0, 0, 0)),
            pl.BlockSpec((C * P * P, E), lambda i: (0, 0)),
            pl.BlockSpec((1, E), lambda i: (0, 0)),
            pl.BlockSpec((1, E), lambda i: (0, 0)),
            pl.BlockSpec((1, E), lambda i: (0, 0)),
            pl.BlockSpec((E, 3 * E), lambda i: (0, 0)),
            pl.BlockSpec((1, 3 * E), lambda i: (0, 0)),
            pl.BlockSpec((E, E), lambda i: (0, 0)),
            pl.BlockSpec((1, E), lambda i: (0, 0)),
            pl.BlockSpec((E, NE), lambda i: (0, 0)),
            pl.BlockSpec((1, NE), lambda i: (0, 0)),
        ],
        out_specs=[
            pl.BlockSpec((1, N, E), lambda i: (i, 0, 0)),
            pl.BlockSpec((1, N, NE), lambda i: (i, 0, 0)),
        ],
        out_shape=[
            jax.ShapeDtypeStruct((B, N, E), f32),
            jax.ShapeDtypeStruct((B, N, NE), f32),
        ],
    )(patches, pos_embed, pwT, row2(patch_b), row2(ln1_g), row2(ln1_b),
      qkvwT, row2(qkv_b), owT, row2(out_b), rwT, row2(router_b))

    d0, d1, ept = pl.pallas_call(
        _route_body,
        grid=(1,),
        in_specs=[pl.BlockSpec((T, NE), lambda i: (0, 0))],
        out_specs=[
            pl.BlockSpec((T, 1), lambda i: (0, 0)),
            pl.BlockSpec((T, 1), lambda i: (0, 0)),
            pl.BlockSpec((NT, 1), lambda i: (0, 0)),
        ],
        out_shape=[
            jax.ShapeDtypeStruct((T, 1), jnp.int32),
            jax.ShapeDtypeStruct((T, 1), jnp.int32),
            jax.ShapeDtypeStruct((NT, 1), jnp.int32),
        ],
    )(logits.reshape(T, NE))
    ept = ept.reshape(NT)

    z_flat = z.reshape(T, E)
    destE2 = d0.reshape(T)
    destO2 = d1.reshape(T)
    zs = _sc_disperse(z_flat, destE2, destO2)

    w1b = eW1.astype(jnp.bfloat16)
    w2b = eW2.astype(jnp.bfloat16)
    ys = pl.pallas_call(
        _moe_body,
        grid_spec=pltpu.PrefetchScalarGridSpec(
            num_scalar_prefetch=1,
            grid=(NT,),
            in_specs=[
                pl.BlockSpec((TILE, E), lambda t, ept: (t, 0)),
                pl.BlockSpec((1, HID, E), lambda t, ept: (ept[t], 0, 0)),
                pl.BlockSpec((1, 1, HID), lambda t, ept: (ept[t], 0, 0)),
                pl.BlockSpec((1, E, HID), lambda t, ept: (ept[t], 0, 0)),
                pl.BlockSpec((1, 1, E), lambda t, ept: (ept[t], 0, 0)),
            ],
            out_specs=pl.BlockSpec((TILE, E), lambda t, ept: (t, 0)),
        ),
        out_shape=jax.ShapeDtypeStruct((S, E), f32),
    )(ept, zs, w1b, eb1.reshape(NE, 1, HID), w2b, eb2.reshape(NE, 1, E))

    ye, yo = _sc_combine(ys, destE2, destO2)
    ysE = ye.reshape(B, N, E)
    ysO = yo.reshape(B, N, E)
    out = pl.pallas_call(
        _head_body,
        grid=(B // IPB,),
        in_specs=[
            pl.BlockSpec((IPB, N, E), lambda i: (i, 0, 0)),
            pl.BlockSpec((IPB, N, E), lambda i: (i, 0, 0)),
            pl.BlockSpec((1, E), lambda i: (0, 0)),
            pl.BlockSpec((1, E), lambda i: (0, 0)),
            pl.BlockSpec((E, NC), lambda i: (0, 0)),
            pl.BlockSpec((1, NC), lambda i: (0, 0)),
        ],
        out_specs=pl.BlockSpec((IPB, NC), lambda i: (i, 0)),
        out_shape=jax.ShapeDtypeStruct((B, NC), f32),
    )(ysE, ysO, row2(ln2_g), row2(ln2_b), hwT, row2(head_b))
    return out
